# Initial kernel scaffold; baseline (speedup 1.0000x reference)
import jax
import jax.numpy as jnp
from jax.experimental import pallas as pl


def kernel(x, edge_index, params):
    raise NotImplementedError("write your pallas kernel here")


# speedup vs baseline: 5.8931x; 5.8931x over previous
import functools

import jax
import jax.numpy as jnp
from jax import lax
from jax.experimental import pallas as pl
from jax.experimental.pallas import tpu as pltpu
from jax.experimental.pallas import tpu_sc as plsc

_N = 10000
_E = 320000
_H = 128
_NCORES = 2
_NSUB = 16
_NW = _NCORES * _NSUB
_CHUNK = 128
_NCHUNKS = _E // _CHUNK
_CHUNKS_PER_W = -(-_NCHUNKS // _NW)
_RPT = 624

_mesh = plsc.VectorSubcoreMesh(core_axis_name="c", subcore_axis_name="s")


def _zero_fill(ref, nrows, width):
  z = jnp.zeros((16,), jnp.float32)

  def row(i, carry):
    for jj in range(width // 16):
      ref[i, pl.ds(jj * 16, 16)] = z
    return carry

  lax.fori_loop(jnp.int32(0), jnp.int32(nrows), row, 0)


def _one_fill(ref, nrows, width):
  o = jnp.ones((16,), jnp.float32)

  def row(i, carry):
    for jj in range(width // 16):
      ref[i, pl.ds(jj * 16, 16)] = o
    return carry

  lax.fori_loop(jnp.int32(0), jnp.int32(nrows), row, 0)


def _stripe_copy_out(sh_ref, buf_ref, out_ref, c, s, base):
  for k in range(4):
    pltpu.sync_copy(sh_ref.at[pl.ds(base + k * 128, 128)], buf_ref)
    pltpu.sync_copy(buf_ref, out_ref.at[c, pl.ds(base + k * 128, 128)])
  pltpu.sync_copy(sh_ref.at[pl.ds(base + 512, 112)], buf_ref.at[pl.ds(0, 112)])
  pltpu.sync_copy(buf_ref.at[pl.ds(0, 112)],
                  out_ref.at[c, pl.ds(base + 512, 112)])

  @pl.when(s == _NSUB - 1)
  def _():
    pltpu.sync_copy(sh_ref.at[pl.ds(_NSUB * _RPT, 16)], buf_ref.at[pl.ds(0, 16)])
    pltpu.sync_copy(buf_ref.at[pl.ds(0, 16)],
                    out_ref.at[c, pl.ds(_NSUB * _RPT, 16)])


def _stripe_zero(sh_ref, zbuf_ref, s, base):
  for k in range(4):
    pltpu.sync_copy(zbuf_ref, sh_ref.at[pl.ds(base + k * 128, 128)])
  pltpu.sync_copy(zbuf_ref.at[pl.ds(0, 112)], sh_ref.at[pl.ds(base + 512, 112)])

  @pl.when(s == _NSUB - 1)
  def _():
    pltpu.sync_copy(zbuf_ref.at[pl.ds(0, 16)],
                    sh_ref.at[pl.ds(_NSUB * _RPT, 16)])


def _sc_seg_sum_body(h_hbm, src_hbm, dst_hbm, agg_out,
                     agg_sh, rows_v, src_v, dst_v, sem):
  c = lax.axis_index("c")
  s = lax.axis_index("s")
  w = s * _NCORES + c
  base = s * _RPT

  _zero_fill(rows_v, _CHUNK, _H)
  _stripe_zero(agg_sh, rows_v, s, base)
  plsc.subcore_barrier()

  def step(j, carry):
    chunk = w + jnp.int32(_NW) * j

    @pl.when(chunk < _NCHUNKS)
    def _():
      pltpu.sync_copy(src_hbm.at[pl.ds(chunk * _CHUNK, _CHUNK)], src_v)
      pltpu.sync_copy(dst_hbm.at[pl.ds(chunk * _CHUNK, _CHUNK)], dst_v)
      pltpu.async_copy(h_hbm.at[src_v], rows_v, sem).wait()
      pltpu.sync_copy(rows_v, agg_sh.at[dst_v], add=True)

    return carry

  lax.fori_loop(jnp.int32(0), jnp.int32(_CHUNKS_PER_W), step, 0)
  plsc.subcore_barrier()

  _stripe_copy_out(agg_sh, rows_v, agg_out, c, s, base)


_sc_seg_sum = pl.kernel(
    _sc_seg_sum_body,
    out_type=jax.ShapeDtypeStruct((_NCORES, _N, _H), jnp.float32),
    mesh=_mesh,
    scratch_types=[
        pltpu.VMEM_SHARED((_N, _H), jnp.float32),
        pltpu.VMEM((_CHUNK, _H), jnp.float32),
        pltpu.VMEM((_CHUNK,), jnp.int32),
        pltpu.VMEM((_CHUNK,), jnp.int32),
        pltpu.SemaphoreType.DMA,
    ],
)


def _sc_deg_body(dst_hbm, deg_out, deg_sh, ones_v, zb_v, dst_v):
  c = lax.axis_index("c")
  s = lax.axis_index("s")
  w = s * _NCORES + c
  base = s * _RPT

  _zero_fill(zb_v, _CHUNK, _H)
  _one_fill(ones_v, _CHUNK, _H)
  _stripe_zero(deg_sh, zb_v, s, base)
  plsc.subcore_barrier()

  def step(j, carry):
    chunk = w + jnp.int32(_NW) * j

    @pl.when(chunk < _NCHUNKS)
    def _():
      pltpu.sync_copy(dst_hbm.at[pl.ds(chunk * _CHUNK, _CHUNK)], dst_v)
      pltpu.sync_copy(ones_v, deg_sh.at[dst_v], add=True)

    return carry

  lax.fori_loop(jnp.int32(0), jnp.int32(_CHUNKS_PER_W), step, 0)
  plsc.subcore_barrier()

  _stripe_copy_out(deg_sh, zb_v, deg_out, c, s, base)


_sc_deg = pl.kernel(
    _sc_deg_body,
    out_type=jax.ShapeDtypeStruct((_NCORES, _N, _H), jnp.float32),
    mesh=_mesh,
    scratch_types=[
        pltpu.VMEM_SHARED((_N, _H), jnp.float32),
        pltpu.VMEM((_CHUNK, _H), jnp.float32),
        pltpu.VMEM((_CHUNK, _H), jnp.float32),
        pltpu.VMEM((_CHUNK,), jnp.int32),
    ],
)


def _dotT(a, b):
  return lax.dot_general(a, b, (((1,), (1,)), ((), ())),
                         preferred_element_type=jnp.float32)


def _mlp_in_body(x_ref, w_ref, b_ref, o_ref):
  o_ref[...] = jnp.maximum(_dotT(x_ref[...], w_ref[...]) + b_ref[...], 0.0)


_mlp_in = pl.pallas_call(
    _mlp_in_body,
    out_shape=jax.ShapeDtypeStruct((_N, _H), jnp.float32),
)


def _layer_body(p_ref, d_ref, h_ref, wl_ref, bl_ref, wr_ref, g_ref, be_ref,
                o_ref, *, resid):
  deg = d_ref[0, :, 0:1] + d_ref[1, :, 0:1]
  inv = 1.0 / jnp.maximum(deg, 1.0)
  mean = (p_ref[0] + p_ref[1]) * inv
  h = h_ref[...]
  z = _dotT(mean, wl_ref[...]) + bl_ref[...] + _dotT(h, wr_ref[...])
  mu = jnp.mean(z, axis=0, keepdims=True)
  zc = z - mu
  var = jnp.mean(zc * zc, axis=0, keepdims=True)
  zn = g_ref[...] * zc * lax.rsqrt(var + 1e-5) + be_ref[...]
  zr = jnp.maximum(zn, 0.0)
  o_ref[...] = zr + h if resid else zr


_layer_fns = [
    pl.pallas_call(
        functools.partial(_layer_body, resid=(i > 0)),
        out_shape=jax.ShapeDtypeStruct((_N, _H), jnp.float32),
    )
    for i in range(2)
]


def _head_body(h_ref, w1_ref, b1_ref, w2_ref, b2_ref, o_ref):
  a = jnp.maximum(_dotT(h_ref[...], w1_ref[...]) + b1_ref[...], 0.0)
  o_ref[...] = _dotT(a, w2_ref[...]) + b2_ref[...]


_head = pl.pallas_call(
    _head_body,
    out_shape=jax.ShapeDtypeStruct((_N, 8), jnp.float32),
)


def kernel(x, edge_index, params):
  src = edge_index[0].astype(jnp.int32)
  dst = edge_index[1].astype(jnp.int32)
  x = x.astype(jnp.float32)

  in_b = params['in_b'].reshape(1, _H).astype(jnp.float32)
  h = _mlp_in(x, params['in_w'], in_b)

  deg_p = _sc_deg(dst)
  for i in range(3):
    cp = params['convs'][i]
    bp = params['bns'][i]
    agg_p = _sc_seg_sum(h, src, dst)
    h = _layer_fns[1 if i > 0 else 0](
        agg_p, deg_p, h,
        cp['lin_l_w'], cp['lin_l_b'].reshape(1, _H),
        cp['lin_r_w'],
        bp['gamma'].reshape(1, _H), bp['beta'].reshape(1, _H),
    )

  w2 = jnp.zeros((8, 64), jnp.float32).at[:2].set(params['fc2_w'])
  b2 = jnp.zeros((1, 8), jnp.float32).at[0, :2].set(params['fc2_b'])
  out8 = _head(h, params['fc1_w'], params['fc1_b'].reshape(1, 64), w2, b2)
  return (out8[:, :2], h)


# speedup vs baseline: 7.2886x; 1.2368x over previous
import functools

import jax
import jax.numpy as jnp
from jax import lax
from jax.experimental import pallas as pl
from jax.experimental.pallas import tpu as pltpu
from jax.experimental.pallas import tpu_sc as plsc

_N = 10000
_E = 320000
_H = 128
_NCORES = 2
_NSUB = 16
_NW = _NCORES * _NSUB
_CHUNK = 128
_NCHUNKS = _E // _CHUNK
_CHUNKS_PER_W = -(-_NCHUNKS // _NW)
_RPT = 624

_mesh = plsc.VectorSubcoreMesh(core_axis_name="c", subcore_axis_name="s")


def _zero_fill(ref, nrows, width):
  z = jnp.zeros((16,), jnp.float32)

  def row(i, carry):
    for jj in range(width // 16):
      ref[i, pl.ds(jj * 16, 16)] = z
    return carry

  lax.fori_loop(jnp.int32(0), jnp.int32(nrows), row, 0)


def _one_fill(ref, nrows, width):
  o = jnp.ones((16,), jnp.float32)

  def row(i, carry):
    for jj in range(width // 16):
      ref[i, pl.ds(jj * 16, 16)] = o
    return carry

  lax.fori_loop(jnp.int32(0), jnp.int32(nrows), row, 0)


def _stripe_copy_out(sh_ref, buf_ref, out_ref, c, s, base):
  for k in range(4):
    pltpu.sync_copy(sh_ref.at[pl.ds(base + k * 128, 128)], buf_ref)
    pltpu.sync_copy(buf_ref, out_ref.at[c, pl.ds(base + k * 128, 128)])
  pltpu.sync_copy(sh_ref.at[pl.ds(base + 512, 112)], buf_ref.at[pl.ds(0, 112)])
  pltpu.sync_copy(buf_ref.at[pl.ds(0, 112)],
                  out_ref.at[c, pl.ds(base + 512, 112)])

  @pl.when(s == _NSUB - 1)
  def _():
    pltpu.sync_copy(sh_ref.at[pl.ds(_NSUB * _RPT, 16)], buf_ref.at[pl.ds(0, 16)])
    pltpu.sync_copy(buf_ref.at[pl.ds(0, 16)],
                    out_ref.at[c, pl.ds(_NSUB * _RPT, 16)])


def _stripe_zero(sh_ref, zbuf_ref, s, base):
  for k in range(4):
    pltpu.sync_copy(zbuf_ref, sh_ref.at[pl.ds(base + k * 128, 128)])
  pltpu.sync_copy(zbuf_ref.at[pl.ds(0, 112)], sh_ref.at[pl.ds(base + 512, 112)])

  @pl.when(s == _NSUB - 1)
  def _():
    pltpu.sync_copy(zbuf_ref.at[pl.ds(0, 16)],
                    sh_ref.at[pl.ds(_NSUB * _RPT, 16)])


_GRP = 2
_HALF = 1
_NGRP = 39
_CPW = 78
_TAIL0 = _NW * _CPW


def _sc_seg_sum_body(h_hbm, src_hbm, dst_hbm, agg_out, agg_sh,
                     rA0, rB0,
                     srcA, srcB, dstA, dstB,
                     gsemA, gsemB, ssemA, ssemB):
  c = lax.axis_index("c")
  s = lax.axis_index("s")
  w = s * _NCORES + c
  base = s * _RPT
  rowsA = (rA0,)
  rowsB = (rB0,)

  _zero_fill(rA0, _CHUNK, _H)
  _stripe_zero(agg_sh, rA0, s, base)
  plsc.subcore_barrier()

  start = w * _CPW

  def load_idx(src_blk, dst_blk, c0):
    pltpu.sync_copy(src_hbm.at[pl.ds(c0 * _CHUNK, _HALF * _CHUNK)], src_blk)
    pltpu.sync_copy(dst_hbm.at[pl.ds(c0 * _CHUNK, _HALF * _CHUNK)], dst_blk)

  def issue_gathers(rows, src_blk, gsem):
    for r in range(_HALF):
      pltpu.async_copy(h_hbm.at[src_blk.at[pl.ds(r * _CHUNK, _CHUNK)]],
                       rows[r], gsem)

  def issue_scatters(rows, dst_blk, ssem):
    for r in range(_HALF):
      pltpu.async_copy(rows[r],
                       agg_sh.at[dst_blk.at[pl.ds(r * _CHUNK, _CHUNK)]],
                       ssem, add=True)

  def drain(rows, sem):
    for r in range(_HALF):
      pltpu.make_async_copy(h_hbm.at[pl.ds(0, _CHUNK)], rows[r], sem).wait()

  load_idx(srcA, dstA, jnp.int32(start))
  issue_gathers(rowsA, srcA, gsemA)

  def step(t, carry):
    c6 = jnp.int32(start) + jnp.int32(_GRP) * t
    drain(rowsA, gsemA)
    issue_scatters(rowsA, dstA, ssemA)

    @pl.when(t > 0)
    def _():
      drain(rowsB, ssemB)

    load_idx(srcB, dstB, c6 + _HALF)
    issue_gathers(rowsB, srcB, gsemB)
    drain(rowsB, gsemB)
    issue_scatters(rowsB, dstB, ssemB)
    drain(rowsA, ssemA)

    @pl.when(t < _NGRP - 1)
    def _():
      load_idx(srcA, dstA, c6 + _GRP)
      issue_gathers(rowsA, srcA, gsemA)

    return carry

  lax.fori_loop(jnp.int32(0), jnp.int32(_NGRP), step, 0)
  drain(rowsB, ssemB)

  @pl.when(w < _NCHUNKS - _TAIL0)
  def _():
    cc = jnp.int32(_TAIL0) + w
    pltpu.sync_copy(src_hbm.at[pl.ds(cc * _CHUNK, _CHUNK)],
                    srcA.at[pl.ds(0, _CHUNK)])
    pltpu.sync_copy(dst_hbm.at[pl.ds(cc * _CHUNK, _CHUNK)],
                    dstA.at[pl.ds(0, _CHUNK)])
    pltpu.async_copy(h_hbm.at[srcA.at[pl.ds(0, _CHUNK)]], rA0, gsemA).wait()
    pltpu.sync_copy(rA0, agg_sh.at[dstA.at[pl.ds(0, _CHUNK)]], add=True)

  plsc.subcore_barrier()
  _stripe_copy_out(agg_sh, rA0, agg_out, c, s, base)


_sc_seg_sum = pl.kernel(
    _sc_seg_sum_body,
    out_type=jax.ShapeDtypeStruct((_NCORES, _N, _H), jnp.float32),
    mesh=_mesh,
    scratch_types=(
        [pltpu.VMEM_SHARED((_N, _H), jnp.float32)]
        + [pltpu.VMEM((_CHUNK, _H), jnp.float32)] * 2
        + [pltpu.VMEM((_HALF * _CHUNK,), jnp.int32)] * 4
        + [pltpu.SemaphoreType.DMA] * 4
    ),
)


def _sc_deg_body(dst_hbm, deg_out, deg_sh, ones_v, dstA, dstB, ssemA, ssemB):
  c = lax.axis_index("c")
  s = lax.axis_index("s")
  w = s * _NCORES + c
  base = s * _RPT

  _zero_fill(ones_v, _CHUNK, _H)
  _stripe_zero(deg_sh, ones_v, s, base)
  _one_fill(ones_v, _CHUNK, _H)
  plsc.subcore_barrier()

  start = w * _CPW

  def issue_scatters(dst_blk, ssem):
    for r in range(_HALF):
      pltpu.async_copy(ones_v,
                       deg_sh.at[dst_blk.at[pl.ds(r * _CHUNK, _CHUNK)]],
                       ssem, add=True)

  def drain(sem):
    for _ in range(_HALF):
      pltpu.make_async_copy(deg_out.at[jnp.int32(0), pl.ds(0, _CHUNK)],
                            ones_v, sem).wait()

  pltpu.sync_copy(dst_hbm.at[pl.ds(jnp.int32(start) * _CHUNK,
                                   _HALF * _CHUNK)], dstA)

  def step(t, carry):
    c6 = jnp.int32(start) + jnp.int32(_GRP) * t
    issue_scatters(dstA, ssemA)

    @pl.when(t > 0)
    def _():
      drain(ssemB)

    pltpu.sync_copy(dst_hbm.at[pl.ds((c6 + _HALF) * _CHUNK,
                                     _HALF * _CHUNK)], dstB)
    issue_scatters(dstB, ssemB)
    drain(ssemA)

    @pl.when(t < _NGRP - 1)
    def _():
      pltpu.sync_copy(dst_hbm.at[pl.ds((c6 + _GRP) * _CHUNK,
                                       _HALF * _CHUNK)], dstA)

    return carry

  lax.fori_loop(jnp.int32(0), jnp.int32(_NGRP), step, 0)
  drain(ssemB)

  @pl.when(w < _NCHUNKS - _TAIL0)
  def _():
    cc = jnp.int32(_TAIL0) + w
    pltpu.sync_copy(dst_hbm.at[pl.ds(cc * _CHUNK, _CHUNK)],
                    dstA.at[pl.ds(0, _CHUNK)])
    pltpu.sync_copy(ones_v, deg_sh.at[dstA.at[pl.ds(0, _CHUNK)]], add=True)

  plsc.subcore_barrier()
  _stripe_copy_out(deg_sh, ones_v, deg_out, c, s, base)


_sc_deg = pl.kernel(
    _sc_deg_body,
    out_type=jax.ShapeDtypeStruct((_NCORES, _N, _H), jnp.float32),
    mesh=_mesh,
    scratch_types=[
        pltpu.VMEM_SHARED((_N, _H), jnp.float32),
        pltpu.VMEM((_CHUNK, _H), jnp.float32),
        pltpu.VMEM((_HALF * _CHUNK,), jnp.int32),
        pltpu.VMEM((_HALF * _CHUNK,), jnp.int32),
        pltpu.SemaphoreType.DMA,
        pltpu.SemaphoreType.DMA,
    ],
)


def _dotT(a, b):
  return lax.dot_general(a, b, (((1,), (1,)), ((), ())),
                         preferred_element_type=jnp.float32)


def _mlp_in_body(x_ref, w_ref, b_ref, o_ref):
  o_ref[...] = jnp.maximum(_dotT(x_ref[...], w_ref[...]) + b_ref[...], 0.0)


_mlp_in = pl.pallas_call(
    _mlp_in_body,
    out_shape=jax.ShapeDtypeStruct((_N, _H), jnp.float32),
)


def _layer_body(p_ref, d_ref, h_ref, wl_ref, bl_ref, wr_ref, g_ref, be_ref,
                o_ref, *, resid):
  deg = d_ref[0, :, 0:1] + d_ref[1, :, 0:1]
  inv = 1.0 / jnp.maximum(deg, 1.0)
  mean = (p_ref[0] + p_ref[1]) * inv
  h = h_ref[...]
  z = _dotT(mean, wl_ref[...]) + bl_ref[...] + _dotT(h, wr_ref[...])
  mu = jnp.mean(z, axis=0, keepdims=True)
  zc = z - mu
  var = jnp.mean(zc * zc, axis=0, keepdims=True)
  zn = g_ref[...] * zc * lax.rsqrt(var + 1e-5) + be_ref[...]
  zr = jnp.maximum(zn, 0.0)
  o_ref[...] = zr + h if resid else zr


_layer_fns = [
    pl.pallas_call(
        functools.partial(_layer_body, resid=(i > 0)),
        out_shape=jax.ShapeDtypeStruct((_N, _H), jnp.float32),
    )
    for i in range(2)
]


def _head_body(h_ref, w1_ref, b1_ref, w2_ref, b2_ref, o_ref):
  a = jnp.maximum(_dotT(h_ref[...], w1_ref[...]) + b1_ref[...], 0.0)
  o_ref[...] = _dotT(a, w2_ref[...]) + b2_ref[...]


_head = pl.pallas_call(
    _head_body,
    out_shape=jax.ShapeDtypeStruct((_N, 8), jnp.float32),
)


def kernel(x, edge_index, params):
  src = edge_index[0].astype(jnp.int32)
  dst = edge_index[1].astype(jnp.int32)
  x = x.astype(jnp.float32)

  in_b = params['in_b'].reshape(1, _H).astype(jnp.float32)
  h = _mlp_in(x, params['in_w'], in_b)

  deg_p = _sc_deg(dst)
  for i in range(3):
    cp = params['convs'][i]
    bp = params['bns'][i]
    agg_p = _sc_seg_sum(h, src, dst)
    h = _layer_fns[1 if i > 0 else 0](
        agg_p, deg_p, h,
        cp['lin_l_w'], cp['lin_l_b'].reshape(1, _H),
        cp['lin_r_w'],
        bp['gamma'].reshape(1, _H), bp['beta'].reshape(1, _H),
    )

  w2 = jnp.zeros((8, 64), jnp.float32).at[:2].set(params['fc2_w'])
  b2 = jnp.zeros((1, 8), jnp.float32).at[0, :2].set(params['fc2_b'])
  out8 = _head(h, params['fc1_w'], params['fc1_b'].reshape(1, 64), w2, b2)
  return (out8[:, :2], h)


# speedup vs baseline: 9.4569x; 1.2975x over previous
import functools

import jax
import jax.numpy as jnp
from jax import lax
from jax.experimental import pallas as pl
from jax.experimental.pallas import tpu as pltpu
from jax.experimental.pallas import tpu_sc as plsc

_N = 10000
_E = 320000
_H = 128
_NCORES = 2
_NSUB = 16
_NW = _NCORES * _NSUB
_CHUNK = 104
_EPW = _E // _NW
_NCHW = 96
_TAILE = _EPW - _NCHW * _CHUNK
_RPT = 624

_mesh = plsc.VectorSubcoreMesh(core_axis_name="c", subcore_axis_name="s")


def _zero_fill(ref, nrows, width):
  z = jnp.zeros((16,), jnp.float32)

  def row(i, carry):
    for jj in range(width // 16):
      ref[i, pl.ds(jj * 16, 16)] = z
    return carry

  lax.fori_loop(jnp.int32(0), jnp.int32(nrows), row, 0)


def _one_fill(ref, nrows, width):
  o = jnp.ones((16,), jnp.float32)

  def row(i, carry):
    for jj in range(width // 16):
      ref[i, pl.ds(jj * 16, 16)] = o
    return carry

  lax.fori_loop(jnp.int32(0), jnp.int32(nrows), row, 0)


def _stripe_copy_out(sh_ref, buf_ref, out_ref, c, s, base):
  for k in range(6):
    pltpu.sync_copy(sh_ref.at[pl.ds(base + k * _CHUNK, _CHUNK)], buf_ref)
    pltpu.sync_copy(buf_ref, out_ref.at[c, pl.ds(base + k * _CHUNK, _CHUNK)])

  @pl.when(s == _NSUB - 1)
  def _():
    pltpu.sync_copy(sh_ref.at[pl.ds(_NSUB * _RPT, 16)], buf_ref.at[pl.ds(0, 16)])
    pltpu.sync_copy(buf_ref.at[pl.ds(0, 16)],
                    out_ref.at[c, pl.ds(_NSUB * _RPT, 16)])


def _stripe_zero(sh_ref, zbuf_ref, s, base):
  for k in range(6):
    pltpu.sync_copy(zbuf_ref, sh_ref.at[pl.ds(base + k * _CHUNK, _CHUNK)])

  @pl.when(s == _NSUB - 1)
  def _():
    pltpu.sync_copy(zbuf_ref.at[pl.ds(0, 16)],
                    sh_ref.at[pl.ds(_NSUB * _RPT, 16)])


_NGRP = _NCHW // 2


def _sc_seg_sum_body(h_hbm, src_hbm, dst_hbm, agg_out, agg_sh,
                     rA, rB, src_all, dst_all,
                     gsemA, gsemB, ssemA, ssemB):
  c = lax.axis_index("c")
  s = lax.axis_index("s")
  w = s * _NCORES + c
  base = s * _RPT

  _zero_fill(rA, _CHUNK, _H)
  _stripe_zero(agg_sh, rA, s, base)
  plsc.subcore_barrier()

  e0 = w * _EPW
  pltpu.sync_copy(src_hbm.at[pl.ds(e0, _NCHW * _CHUNK)], src_all)
  pltpu.sync_copy(dst_hbm.at[pl.ds(e0, _NCHW * _CHUNK)], dst_all)

  def gather(rows, k, gsem):
    pltpu.async_copy(h_hbm.at[src_all.at[pl.ds(k * _CHUNK, _CHUNK)]],
                     rows, gsem)

  def scatter(rows, k, ssem):
    pltpu.async_copy(rows, agg_sh.at[dst_all.at[pl.ds(k * _CHUNK, _CHUNK)]],
                     ssem, add=True)

  def drain(rows, sem):
    pltpu.make_async_copy(h_hbm.at[pl.ds(0, _CHUNK)], rows, sem).wait()

  gather(rA, jnp.int32(0), gsemA)

  def step(t, carry):
    k = jnp.int32(2) * t
    drain(rA, gsemA)
    scatter(rA, k, ssemA)

    @pl.when(t > 0)
    def _():
      drain(rB, ssemB)

    gather(rB, k + 1, gsemB)
    drain(rB, gsemB)
    scatter(rB, k + 1, ssemB)
    drain(rA, ssemA)

    @pl.when(t < _NGRP - 1)
    def _():
      gather(rA, k + 2, gsemA)

    return carry

  lax.fori_loop(jnp.int32(0), jnp.int32(_NGRP), step, 0)
  drain(rB, ssemB)

  pltpu.sync_copy(src_hbm.at[pl.ds(e0 + _NCHW * _CHUNK, _TAILE)],
                  src_all.at[pl.ds(0, _TAILE)])
  pltpu.sync_copy(dst_hbm.at[pl.ds(e0 + _NCHW * _CHUNK, _TAILE)],
                  dst_all.at[pl.ds(0, _TAILE)])
  pltpu.async_copy(h_hbm.at[src_all.at[pl.ds(0, _TAILE)]],
                   rA.at[pl.ds(0, _TAILE)], gsemA).wait()
  pltpu.sync_copy(rA.at[pl.ds(0, _TAILE)],
                  agg_sh.at[dst_all.at[pl.ds(0, _TAILE)]], add=True)

  plsc.subcore_barrier()
  _stripe_copy_out(agg_sh, rA, agg_out, c, s, base)


_sc_seg_sum = pl.kernel(
    _sc_seg_sum_body,
    out_type=jax.ShapeDtypeStruct((_NCORES, _N, _H), jnp.float32),
    mesh=_mesh,
    scratch_types=(
        [pltpu.VMEM_SHARED((_N, _H), jnp.float32)]
        + [pltpu.VMEM((_CHUNK, _H), jnp.float32)] * 2
        + [pltpu.VMEM((_NCHW * _CHUNK,), jnp.int32)] * 2
        + [pltpu.SemaphoreType.DMA] * 4
    ),
)


def _sc_deg_body(dst_hbm, deg_out, deg_sh, ones_v, dst_all, ssemA, ssemB):
  c = lax.axis_index("c")
  s = lax.axis_index("s")
  w = s * _NCORES + c
  base = s * _RPT

  _zero_fill(ones_v, _CHUNK, _H)
  _stripe_zero(deg_sh, ones_v, s, base)
  _one_fill(ones_v, _CHUNK, _H)
  plsc.subcore_barrier()

  e0 = w * _EPW
  pltpu.sync_copy(dst_hbm.at[pl.ds(e0, _NCHW * _CHUNK)], dst_all)

  def scatter(k, ssem):
    pltpu.async_copy(ones_v,
                     deg_sh.at[dst_all.at[pl.ds(k * _CHUNK, _CHUNK)]],
                     ssem, add=True)

  def drain(sem):
    pltpu.make_async_copy(deg_out.at[jnp.int32(0), pl.ds(0, _CHUNK)],
                          ones_v, sem).wait()

  def step(t, carry):
    k = jnp.int32(2) * t

    @pl.when(t > 0)
    def _():
      drain(ssemA)
      drain(ssemB)

    scatter(k, ssemA)
    scatter(k + 1, ssemB)
    return carry

  lax.fori_loop(jnp.int32(0), jnp.int32(_NGRP), step, 0)
  drain(ssemA)
  drain(ssemB)

  pltpu.sync_copy(dst_hbm.at[pl.ds(e0 + _NCHW * _CHUNK, _TAILE)],
                  dst_all.at[pl.ds(0, _TAILE)])
  pltpu.sync_copy(ones_v.at[pl.ds(0, _TAILE)],
                  deg_sh.at[dst_all.at[pl.ds(0, _TAILE)]], add=True)

  plsc.subcore_barrier()
  _stripe_copy_out(deg_sh, ones_v, deg_out, c, s, base)


_sc_deg = pl.kernel(
    _sc_deg_body,
    out_type=jax.ShapeDtypeStruct((_NCORES, _N, _H), jnp.float32),
    mesh=_mesh,
    scratch_types=[
        pltpu.VMEM_SHARED((_N, _H), jnp.float32),
        pltpu.VMEM((_CHUNK, _H), jnp.float32),
        pltpu.VMEM((_NCHW * _CHUNK,), jnp.int32),
        pltpu.SemaphoreType.DMA,
        pltpu.SemaphoreType.DMA,
    ],
)


def _dotT(a, b):
  return lax.dot_general(a, b, (((1,), (1,)), ((), ())),
                         preferred_element_type=jnp.float32)


def _mlp_in_body(x_ref, w_ref, b_ref, o_ref):
  o_ref[...] = jnp.maximum(_dotT(x_ref[...], w_ref[...]) + b_ref[...], 0.0)


_mlp_in = pl.pallas_call(
    _mlp_in_body,
    out_shape=jax.ShapeDtypeStruct((_N, _H), jnp.float32),
)


def _layer_body(p_ref, d_ref, h_ref, wl_ref, bl_ref, wr_ref, g_ref, be_ref,
                o_ref, *, resid):
  deg = d_ref[0, :, 0:1] + d_ref[1, :, 0:1]
  inv = 1.0 / jnp.maximum(deg, 1.0)
  mean = (p_ref[0] + p_ref[1]) * inv
  h = h_ref[...]
  z = _dotT(mean, wl_ref[...]) + bl_ref[...] + _dotT(h, wr_ref[...])
  mu = jnp.mean(z, axis=0, keepdims=True)
  zc = z - mu
  var = jnp.mean(zc * zc, axis=0, keepdims=True)
  zn = g_ref[...] * zc * lax.rsqrt(var + 1e-5) + be_ref[...]
  zr = jnp.maximum(zn, 0.0)
  o_ref[...] = zr + h if resid else zr


_layer_fns = [
    pl.pallas_call(
        functools.partial(_layer_body, resid=(i > 0)),
        out_shape=jax.ShapeDtypeStruct((_N, _H), jnp.float32),
    )
    for i in range(2)
]


def _head_body(h_ref, w1_ref, b1_ref, w2_ref, b2_ref, o_ref):
  a = jnp.maximum(_dotT(h_ref[...], w1_ref[...]) + b1_ref[...], 0.0)
  o_ref[...] = _dotT(a, w2_ref[...]) + b2_ref[...]


_head = pl.pallas_call(
    _head_body,
    out_shape=jax.ShapeDtypeStruct((_N, 8), jnp.float32),
)


def kernel(x, edge_index, params):
  src = edge_index[0].astype(jnp.int32)
  dst = edge_index[1].astype(jnp.int32)
  x = x.astype(jnp.float32)

  in_b = params['in_b'].reshape(1, _H).astype(jnp.float32)
  h = _mlp_in(x, params['in_w'], in_b)

  deg_p = _sc_deg(dst)
  for i in range(3):
    cp = params['convs'][i]
    bp = params['bns'][i]
    agg_p = _sc_seg_sum(h, src, dst)
    h = _layer_fns[1 if i > 0 else 0](
        agg_p, deg_p, h,
        cp['lin_l_w'], cp['lin_l_b'].reshape(1, _H),
        cp['lin_r_w'],
        bp['gamma'].reshape(1, _H), bp['beta'].reshape(1, _H),
    )

  w2 = jnp.zeros((8, 64), jnp.float32).at[:2].set(params['fc2_w'])
  b2 = jnp.zeros((1, 8), jnp.float32).at[0, :2].set(params['fc2_b'])
  out8 = _head(h, params['fc1_w'], params['fc1_b'].reshape(1, 64), w2, b2)
  return (out8[:, :2], h)


# speedup vs baseline: 10.7028x; 1.1318x over previous
import functools

import jax
import jax.numpy as jnp
from jax import lax
from jax.experimental import pallas as pl
from jax.experimental.pallas import tpu as pltpu
from jax.experimental.pallas import tpu_sc as plsc

_N = 10000
_E = 320000
_H = 128
_NCORES = 2
_NSUB = 16
_NW = _NCORES * _NSUB
_CHUNK = 104
_EPW = _E // _NW
_NCHW = 96
_TAILE = _EPW - _NCHW * _CHUNK
_RPT = 624

_mesh = plsc.VectorSubcoreMesh(core_axis_name="c", subcore_axis_name="s")


def _zero_fill(ref, nrows, width):
  z = jnp.zeros((16,), jnp.float32)

  def row(i, carry):
    for jj in range(width // 16):
      ref[i, pl.ds(jj * 16, 16)] = z
    return carry

  lax.fori_loop(jnp.int32(0), jnp.int32(nrows), row, 0)


def _one_fill(ref, nrows, width):
  o = jnp.ones((16,), jnp.float32)

  def row(i, carry):
    for jj in range(width // 16):
      ref[i, pl.ds(jj * 16, 16)] = o
    return carry

  lax.fori_loop(jnp.int32(0), jnp.int32(nrows), row, 0)


def _stripe_copy_out(sh_ref, buf_ref, out_ref, c, s, base):
  for k in range(6):
    pltpu.sync_copy(sh_ref.at[pl.ds(base + k * _CHUNK, _CHUNK)], buf_ref)
    pltpu.sync_copy(buf_ref, out_ref.at[c, pl.ds(base + k * _CHUNK, _CHUNK)])

  @pl.when(s == _NSUB - 1)
  def _():
    pltpu.sync_copy(sh_ref.at[pl.ds(_NSUB * _RPT, 16)], buf_ref.at[pl.ds(0, 16)])
    pltpu.sync_copy(buf_ref.at[pl.ds(0, 16)],
                    out_ref.at[c, pl.ds(_NSUB * _RPT, 16)])


def _stripe_zero(sh_ref, zbuf_ref, s, base):
  for k in range(6):
    pltpu.sync_copy(zbuf_ref, sh_ref.at[pl.ds(base + k * _CHUNK, _CHUNK)])

  @pl.when(s == _NSUB - 1)
  def _():
    pltpu.sync_copy(zbuf_ref.at[pl.ds(0, 16)],
                    sh_ref.at[pl.ds(_NSUB * _RPT, 16)])


_NGRP = _NCHW // 2


def _sc_seg_sum_body(h_hbm, src_hbm, dst_hbm, agg_out, agg_sh,
                     rA, rB, src_all, dst_all,
                     gsemA, gsemB, ssemA, ssemB):
  c = lax.axis_index("c")
  s = lax.axis_index("s")
  w = s * _NCORES + c
  base = s * _RPT

  _zero_fill(rA, _CHUNK, _H)
  _stripe_zero(agg_sh, rA, s, base)
  plsc.subcore_barrier()

  e0 = w * _EPW
  pltpu.sync_copy(src_hbm.at[pl.ds(e0, _NCHW * _CHUNK)], src_all)
  pltpu.sync_copy(dst_hbm.at[pl.ds(e0, _NCHW * _CHUNK)], dst_all)

  def gather(rows, k, gsem):
    pltpu.async_copy(h_hbm.at[src_all.at[pl.ds(k * _CHUNK, _CHUNK)]],
                     rows, gsem)

  def scatter(rows, k, ssem):
    pltpu.async_copy(rows, agg_sh.at[dst_all.at[pl.ds(k * _CHUNK, _CHUNK)]],
                     ssem, add=True)

  def drain(rows, sem):
    pltpu.make_async_copy(h_hbm.at[pl.ds(0, _CHUNK)], rows, sem).wait()

  gather(rA, jnp.int32(0), gsemA)

  def step(t, carry):
    k = jnp.int32(2) * t
    drain(rA, gsemA)
    scatter(rA, k, ssemA)

    @pl.when(t > 0)
    def _():
      drain(rB, ssemB)

    gather(rB, k + 1, gsemB)
    drain(rB, gsemB)
    scatter(rB, k + 1, ssemB)
    drain(rA, ssemA)

    @pl.when(t < _NGRP - 1)
    def _():
      gather(rA, k + 2, gsemA)

    return carry

  lax.fori_loop(jnp.int32(0), jnp.int32(_NGRP), step, 0)
  drain(rB, ssemB)

  pltpu.sync_copy(src_hbm.at[pl.ds(e0 + _NCHW * _CHUNK, _TAILE)],
                  src_all.at[pl.ds(0, _TAILE)])
  pltpu.sync_copy(dst_hbm.at[pl.ds(e0 + _NCHW * _CHUNK, _TAILE)],
                  dst_all.at[pl.ds(0, _TAILE)])
  pltpu.async_copy(h_hbm.at[src_all.at[pl.ds(0, _TAILE)]],
                   rA.at[pl.ds(0, _TAILE)], gsemA).wait()
  pltpu.sync_copy(rA.at[pl.ds(0, _TAILE)],
                  agg_sh.at[dst_all.at[pl.ds(0, _TAILE)]], add=True)

  plsc.subcore_barrier()
  _stripe_copy_out(agg_sh, rA, agg_out, c, s, base)


_sc_seg_sum = pl.kernel(
    _sc_seg_sum_body,
    out_type=jax.ShapeDtypeStruct((_NCORES, _N, _H), jnp.float32),
    mesh=_mesh,
    scratch_types=(
        [pltpu.VMEM_SHARED((_N, _H), jnp.float32)]
        + [pltpu.VMEM((_CHUNK, _H), jnp.float32)] * 2
        + [pltpu.VMEM((_NCHW * _CHUNK,), jnp.int32)] * 2
        + [pltpu.SemaphoreType.DMA] * 4
    ),
)


def _sc_deg_body(dst_hbm, deg_out, hist_v, idx_v):
  c = lax.axis_index("c")
  s = lax.axis_index("s")
  w = s * _NCORES + c
  z = jnp.zeros((16,), jnp.float32)

  def zrow(i, carry):
    hist_v[pl.ds(i * 16, 16)] = z
    return carry

  lax.fori_loop(jnp.int32(0), jnp.int32(_N // 16), zrow, 0)
  pltpu.sync_copy(dst_hbm.at[pl.ds(w * _EPW, _EPW)], idx_v)
  ones = jnp.ones((16,), jnp.float32)

  def step(i, carry):
    iv = idx_v[pl.ds(i * 16, 16)]
    plsc.addupdate_scatter(hist_v, [iv], ones)
    return carry

  lax.fori_loop(jnp.int32(0), jnp.int32(_EPW // 16), step, 0)
  pltpu.sync_copy(hist_v, deg_out.at[c, s])


_sc_deg = pl.kernel(
    _sc_deg_body,
    compiler_params=pltpu.CompilerParams(needs_layout_passes=False),
    out_type=jax.ShapeDtypeStruct((_NCORES, _NSUB, _N), jnp.float32),
    mesh=_mesh,
    scratch_types=[
        pltpu.VMEM((_N,), jnp.float32),
        pltpu.VMEM((_EPW,), jnp.int32),
    ],
)


def _dotT(a, b):
  return lax.dot_general(a, b, (((1,), (1,)), ((), ())),
                         preferred_element_type=jnp.float32)


def _mlp_in_body(x_ref, w_ref, b_ref, o_ref):
  o_ref[...] = jnp.maximum(_dotT(x_ref[...], w_ref[...]) + b_ref[...], 0.0)


_mlp_in = pl.pallas_call(
    _mlp_in_body,
    out_shape=jax.ShapeDtypeStruct((_N, _H), jnp.float32),
)


def _layer_body(p_ref, d_ref, h_ref, wl_ref, bl_ref, wr_ref, g_ref, be_ref,
                *rest, resid, head):
  if head:
    w1_ref, b1_ref, w2_ref, b2_ref, o_ref, o8_ref = rest
  else:
    (o_ref,) = rest
  d = jnp.sum(d_ref[...], axis=0, keepdims=True)
  inv = 1.0 / jnp.maximum(d, 1.0)
  invc = lax.transpose(inv, (1, 0))
  mean = (p_ref[0] + p_ref[1]) * invc
  h = h_ref[...]
  z = _dotT(mean, wl_ref[...]) + bl_ref[...] + _dotT(h, wr_ref[...])
  mu = jnp.mean(z, axis=0, keepdims=True)
  zc = z - mu
  var = jnp.mean(zc * zc, axis=0, keepdims=True)
  zn = g_ref[...] * zc * lax.rsqrt(var + 1e-5) + be_ref[...]
  zr = jnp.maximum(zn, 0.0)
  hn = zr + h if resid else zr
  o_ref[...] = hn
  if head:
    a = jnp.maximum(_dotT(hn, w1_ref[...]) + b1_ref[...], 0.0)
    o8_ref[...] = _dotT(a, w2_ref[...]) + b2_ref[...]


_layer_fns = [
    pl.pallas_call(
        functools.partial(_layer_body, resid=(i > 0), head=(i == 2)),
        out_shape=(
            (jax.ShapeDtypeStruct((_N, _H), jnp.float32),
             jax.ShapeDtypeStruct((_N, 8), jnp.float32))
            if i == 2 else jax.ShapeDtypeStruct((_N, _H), jnp.float32)
        ),
    )
    for i in range(3)
]


def kernel(x, edge_index, params):
  src = edge_index[0].astype(jnp.int32)
  dst = edge_index[1].astype(jnp.int32)
  x = x.astype(jnp.float32)

  in_b = params['in_b'].reshape(1, _H).astype(jnp.float32)
  h = _mlp_in(x, params['in_w'], in_b)

  deg_p = _sc_deg(dst).reshape(_NW, _N)
  w2 = jnp.zeros((8, 64), jnp.float32).at[:2].set(params['fc2_w'])
  b2 = jnp.zeros((1, 8), jnp.float32).at[0, :2].set(params['fc2_b'])

  out8 = None
  for i in range(3):
    cp = params['convs'][i]
    bp = params['bns'][i]
    agg_p = _sc_seg_sum(h, src, dst)
    args = [agg_p, deg_p, h,
            cp['lin_l_w'], cp['lin_l_b'].reshape(1, _H),
            cp['lin_r_w'],
            bp['gamma'].reshape(1, _H), bp['beta'].reshape(1, _H)]
    if i == 2:
      args += [params['fc1_w'], params['fc1_b'].reshape(1, 64), w2, b2]
      h, out8 = _layer_fns[i](*args)
    else:
      h = _layer_fns[i](*args)

  return (out8[:, :2], h)


# speedup vs baseline: 10.7090x; 1.0006x over previous
import functools

import jax
import jax.numpy as jnp
from jax import lax
from jax.experimental import pallas as pl
from jax.experimental.pallas import tpu as pltpu
from jax.experimental.pallas import tpu_sc as plsc

_N = 10000
_E = 320000
_H = 128
_NCORES = 2
_NSUB = 16
_NW = _NCORES * _NSUB
_CHUNK = 104
_EPW = _E // _NW
_NCHW = 96
_TAILE = _EPW - _NCHW * _CHUNK
_RPT = 624

_mesh = plsc.VectorSubcoreMesh(core_axis_name="c", subcore_axis_name="s")


def _zero_fill(ref, nrows, width):
  z = jnp.zeros((16,), jnp.float32)

  def row(i, carry):
    for jj in range(width // 16):
      ref[i, pl.ds(jj * 16, 16)] = z
    return carry

  lax.fori_loop(jnp.int32(0), jnp.int32(nrows), row, 0)


def _stripe_copy_out(sh_ref, buf_ref, out_ref, c, s, base):
  for k in range(6):
    pltpu.sync_copy(sh_ref.at[pl.ds(base + k * _CHUNK, _CHUNK)], buf_ref)
    pltpu.sync_copy(buf_ref, out_ref.at[c, pl.ds(base + k * _CHUNK, _CHUNK)])

  @pl.when(s == _NSUB - 1)
  def _():
    pltpu.sync_copy(sh_ref.at[pl.ds(_NSUB * _RPT, 16)], buf_ref.at[pl.ds(0, 16)])
    pltpu.sync_copy(buf_ref.at[pl.ds(0, 16)],
                    out_ref.at[c, pl.ds(_NSUB * _RPT, 16)])


def _stripe_zero(sh_ref, zbuf_ref, s, base):
  for k in range(6):
    pltpu.sync_copy(zbuf_ref, sh_ref.at[pl.ds(base + k * _CHUNK, _CHUNK)])

  @pl.when(s == _NSUB - 1)
  def _():
    pltpu.sync_copy(zbuf_ref.at[pl.ds(0, 16)],
                    sh_ref.at[pl.ds(_NSUB * _RPT, 16)])


_NGRP = _NCHW // 2


def _sc_seg_sum_body(h_hbm, src_hbm, dst_hbm, agg_out, agg_sh,
                     rA, rB, src_all, dst_all,
                     gsemA, gsemB, ssemA, ssemB):
  c = lax.axis_index("c")
  s = lax.axis_index("s")
  w = s * _NCORES + c
  base = s * _RPT

  _zero_fill(rA, _CHUNK, _H)
  _stripe_zero(agg_sh, rA, s, base)
  plsc.subcore_barrier()

  e0 = w * _EPW
  pltpu.sync_copy(src_hbm.at[pl.ds(e0, _NCHW * _CHUNK)], src_all)
  pltpu.sync_copy(dst_hbm.at[pl.ds(e0, _NCHW * _CHUNK)], dst_all)

  def gather(rows, k, gsem):
    pltpu.async_copy(h_hbm.at[src_all.at[pl.ds(k * _CHUNK, _CHUNK)]],
                     rows, gsem)

  def scatter(rows, k, ssem):
    pltpu.async_copy(rows, agg_sh.at[dst_all.at[pl.ds(k * _CHUNK, _CHUNK)]],
                     ssem, add=True)

  def drain(rows, sem):
    pltpu.make_async_copy(h_hbm.at[pl.ds(0, _CHUNK)], rows, sem).wait()

  gather(rA, jnp.int32(0), gsemA)

  def step(t, carry):
    k = jnp.int32(2) * t
    drain(rA, gsemA)
    scatter(rA, k, ssemA)

    @pl.when(t > 0)
    def _():
      drain(rB, ssemB)

    gather(rB, k + 1, gsemB)
    drain(rB, gsemB)
    scatter(rB, k + 1, ssemB)
    drain(rA, ssemA)

    @pl.when(t < _NGRP - 1)
    def _():
      gather(rA, k + 2, gsemA)

    return carry

  lax.fori_loop(jnp.int32(0), jnp.int32(_NGRP), step, 0)
  drain(rB, ssemB)

  pltpu.sync_copy(src_hbm.at[pl.ds(e0 + _NCHW * _CHUNK, _TAILE)],
                  src_all.at[pl.ds(0, _TAILE)])
  pltpu.sync_copy(dst_hbm.at[pl.ds(e0 + _NCHW * _CHUNK, _TAILE)],
                  dst_all.at[pl.ds(0, _TAILE)])
  pltpu.async_copy(h_hbm.at[src_all.at[pl.ds(0, _TAILE)]],
                   rA.at[pl.ds(0, _TAILE)], gsemA).wait()
  pltpu.sync_copy(rA.at[pl.ds(0, _TAILE)],
                  agg_sh.at[dst_all.at[pl.ds(0, _TAILE)]], add=True)

  plsc.subcore_barrier()
  _stripe_copy_out(agg_sh, rA, agg_out, c, s, base)


_sc_seg_sum = pl.kernel(
    _sc_seg_sum_body,
    out_type=jax.ShapeDtypeStruct((_NCORES, _N, _H), jnp.float32),
    mesh=_mesh,
    scratch_types=(
        [pltpu.VMEM_SHARED((_N, _H), jnp.float32)]
        + [pltpu.VMEM((_CHUNK, _H), jnp.float32)] * 2
        + [pltpu.VMEM((_NCHW * _CHUNK,), jnp.int32)] * 2
        + [pltpu.SemaphoreType.DMA] * 4
    ),
)


def _sc_deg_body(dst_hbm, deg_out, hist_v, idx_v):
  c = lax.axis_index("c")
  s = lax.axis_index("s")
  w = s * _NCORES + c
  z = jnp.zeros((16,), jnp.float32)

  def zrow(i, carry):
    hist_v[pl.ds(i * 16, 16)] = z
    return carry

  lax.fori_loop(jnp.int32(0), jnp.int32(_N // 16), zrow, 0)
  pltpu.sync_copy(dst_hbm.at[pl.ds(w * _EPW, _EPW)], idx_v)
  ones = jnp.ones((16,), jnp.float32)

  def step(i, carry):
    iv = idx_v[pl.ds(i * 16, 16)]
    plsc.addupdate_scatter(hist_v, [iv], ones)
    return carry

  lax.fori_loop(jnp.int32(0), jnp.int32(_EPW // 16), step, 0)
  pltpu.sync_copy(hist_v, deg_out.at[c, s])


_sc_deg = pl.kernel(
    _sc_deg_body,
    compiler_params=pltpu.CompilerParams(needs_layout_passes=False),
    out_type=jax.ShapeDtypeStruct((_NCORES, _NSUB, _N), jnp.float32),
    mesh=_mesh,
    scratch_types=[
        pltpu.VMEM((_N,), jnp.float32),
        pltpu.VMEM((_EPW,), jnp.int32),
    ],
)


def _dotT(a, b):
  return lax.dot_general(a, b, (((1,), (1,)), ((), ())),
                         preferred_element_type=jnp.float32)


def _mlp_in_body(x_ref, w_ref, b_ref, o_ref):
  o_ref[...] = jnp.maximum(_dotT(x_ref[...], w_ref[...]) + b_ref[...], 0.0)


_mlp_in = pl.pallas_call(
    _mlp_in_body,
    out_shape=jax.ShapeDtypeStruct((_N, _H), jnp.float32),
)


def _layer_body(p_ref, d_ref, h_ref, wl_ref, bl_ref, wr_ref, g_ref, be_ref,
                *rest, resid, head):
  if head:
    w1_ref, b1_ref, w2_ref, b2_ref, o_ref, o8_ref = rest
  else:
    (o_ref,) = rest
  d = jnp.sum(d_ref[...], axis=0, keepdims=True)
  inv = 1.0 / jnp.maximum(d, 1.0)
  invc = lax.transpose(inv, (1, 0))
  mean = (p_ref[0] + p_ref[1]) * invc
  h = h_ref[...]
  z = _dotT(mean, wl_ref[...]) + bl_ref[...] + _dotT(h, wr_ref[...])
  mu = jnp.mean(z, axis=0, keepdims=True)
  zc = z - mu
  var = jnp.mean(zc * zc, axis=0, keepdims=True)
  zn = g_ref[...] * zc * lax.rsqrt(var + 1e-5) + be_ref[...]
  zr = jnp.maximum(zn, 0.0)
  hn = zr + h if resid else zr
  o_ref[...] = hn
  if head:
    a = jnp.maximum(_dotT(hn, w1_ref[...]) + b1_ref[...], 0.0)
    o8_ref[...] = _dotT(a, w2_ref[...]) + b2_ref[...]


_layer_fns = [
    pl.pallas_call(
        functools.partial(_layer_body, resid=(i > 0), head=(i == 2)),
        out_shape=(
            (jax.ShapeDtypeStruct((_N, _H), jnp.float32),
             jax.ShapeDtypeStruct((_N, 8), jnp.float32))
            if i == 2 else jax.ShapeDtypeStruct((_N, _H), jnp.float32)
        ),
    )
    for i in range(3)
]


def kernel(x, edge_index, params):
  src = edge_index[0].astype(jnp.int32)
  dst = edge_index[1].astype(jnp.int32)
  x = x.astype(jnp.float32)

  deg_p = _sc_deg(dst).reshape(_NW, _N)

  in_b = params['in_b'].reshape(1, _H).astype(jnp.float32)
  h = _mlp_in(x, params['in_w'], in_b)
  w2 = jnp.zeros((8, 64), jnp.float32).at[:2].set(params['fc2_w'])
  b2 = jnp.zeros((1, 8), jnp.float32).at[0, :2].set(params['fc2_b'])

  out8 = None
  for i in range(3):
    cp = params['convs'][i]
    bp = params['bns'][i]
    agg_p = _sc_seg_sum(h, src, dst)
    args = [agg_p, deg_p, h,
            cp['lin_l_w'], cp['lin_l_b'].reshape(1, _H),
            cp['lin_r_w'],
            bp['gamma'].reshape(1, _H), bp['beta'].reshape(1, _H)]
    if i == 2:
      args += [params['fc1_w'], params['fc1_b'].reshape(1, 64), w2, b2]
      h, out8 = _layer_fns[i](*args)
    else:
      h = _layer_fns[i](*args)

  return (out8[:, :2], h)
